# Initial kernel scaffold; baseline (speedup 1.0000x reference)
import jax
import jax.numpy as jnp
from jax.experimental import pallas as pl


def kernel(x, edge_index, h, c, i, f, W_lin, b_node, Wl1, Wr1, b1, Wl2, Wr2, b2):
    raise NotImplementedError("write your pallas kernel here")


# speedup vs baseline: 6.1258x; 6.1258x over previous
import functools

import jax
import jax.numpy as jnp
from jax import lax
from jax.experimental import pallas as pl
from jax.experimental.pallas import tpu as pltpu
from jax.experimental.pallas import tpu_sc as plsc

D = 128
NC = 2
NS = 16
NW = NC * NS
CH = 128
ROWB = 1024


def _make_seg_sum(n, e, with_cnt):
    epw = e // NW
    n_full = epw // CH
    tail = epw - n_full * CH
    rpt = (-(-n // NS) + 15) // 16 * 16
    npad = rpt * NS
    nz = rpt // 16

    mesh = plsc.VectorSubcoreMesh(
        core_axis_name="c", subcore_axis_name="s",
        num_cores=NC, num_subcores=NS)

    out_type = [jax.ShapeDtypeStruct((NC, npad, D), jnp.float32)]
    scratch = [
        pltpu.VMEM((CH,), jnp.int32),
        pltpu.VMEM((CH,), jnp.int32),
        pltpu.VMEM((CH, D), jnp.float32),
        pltpu.VMEM((16,), jnp.int32),
        pltpu.VMEM((16,), jnp.int32),
        pltpu.VMEM((16, D), jnp.float32),
        pltpu.VMEM((16, D), jnp.float32),
        pltpu.VMEM_SHARED((npad, D), jnp.float32),
        pltpu.SemaphoreType.DMA,
    ]
    if with_cnt:
        out_type.append(jax.ShapeDtypeStruct((NW, npad), jnp.float32))
        scratch.append(pltpu.VMEM((npad,), jnp.float32))

    @functools.partial(
        pl.kernel, out_type=tuple(out_type), mesh=mesh,
        scratch_types=tuple(scratch),
        compiler_params=pltpu.CompilerParams(needs_layout_passes=False))
    def seg_sum(t_hbm, src_hbm, dst_hbm, out_hbm, *rest):
        if with_cnt:
            cnt_hbm, sidx, didx, rows, sidx_t, didx_t, rows_t, zbuf, acc, \
                sem, cnt_v = rest
        else:
            sidx, didx, rows, sidx_t, didx_t, rows_t, zbuf, acc, sem = rest
        cid = lax.axis_index("c")
        sid = lax.axis_index("s")
        wid = cid * NS + sid

        zero16 = jnp.zeros((16,), jnp.float32)
        for r in range(16):
            for cc in range(D // 16):
                zbuf[r, pl.ds(cc * 16, 16)] = zero16

        zbase = sid * rpt

        def zloop(j, carry):
            pltpu.sync_copy(zbuf, acc.at[pl.ds(zbase + j * 16, 16)])
            return carry

        lax.fori_loop(0, nz, zloop, 0)

        if with_cnt:
            def czloop(j, carry):
                cnt_v[pl.ds(j * 16, 16)] = zero16
                return carry
            lax.fori_loop(0, npad // 16, czloop, 0)
            ones16 = jnp.ones((16,), jnp.float32)

        plsc.subcore_barrier()

        ebase = wid * epw

        def accum_cnt(idx_ref, m):
            for k in range(m // 16):
                idx16 = idx_ref[pl.ds(k * 16, 16)]
                plsc.addupdate_scatter(cnt_v, [idx16], ones16)

        def body(j, carry):
            off = ebase + j * CH
            pltpu.sync_copy(src_hbm.at[pl.ds(off, CH)], sidx)
            pltpu.sync_copy(dst_hbm.at[pl.ds(off, CH)], didx)
            pltpu.async_copy(t_hbm.at[sidx], rows, sem).wait()
            pltpu.sync_copy(rows, acc.at[didx], add=True)
            if with_cnt:
                accum_cnt(didx, CH)
            return carry

        lax.fori_loop(0, n_full, body, 0)
        if tail:
            off = ebase + n_full * CH
            pltpu.sync_copy(src_hbm.at[pl.ds(off, tail)], sidx_t)
            pltpu.sync_copy(dst_hbm.at[pl.ds(off, tail)], didx_t)
            pltpu.async_copy(t_hbm.at[sidx_t], rows_t, sem).wait()
            pltpu.sync_copy(rows_t, acc.at[didx_t], add=True)
            if with_cnt:
                accum_cnt(didx_t, tail)

        plsc.subcore_barrier()
        pltpu.sync_copy(acc.at[pl.ds(zbase, rpt)],
                        out_hbm.at[cid, pl.ds(zbase, rpt)])
        if with_cnt:
            pltpu.sync_copy(cnt_v, cnt_hbm.at[wid])

    return seg_sum, npad


def _row_spec(w):
    return pl.BlockSpec((ROWB, w), lambda i: (i, 0))


def _full_spec(shape):
    return pl.BlockSpec(shape, lambda i: tuple(0 for _ in shape))


def _lin0_body(x_ref, w_ref, o_ref):
    o_ref[...] = jnp.dot(x_ref[...], w_ref[...],
                         preferred_element_type=jnp.float32)


def _sage_body(a0_ref, a1_ref, cnt_ref, t_ref, wl_ref, wr_ref, b_ref, o_ref):
    s = a0_ref[...] + a1_ref[...]
    cnt = jnp.maximum(jnp.sum(cnt_ref[...], axis=0), 1.0)[:, None]
    mean = s / cnt
    o_ref[...] = (jnp.dot(mean, wl_ref[...], preferred_element_type=jnp.float32)
                  + jnp.dot(t_ref[...], wr_ref[...],
                            preferred_element_type=jnp.float32)
                  + b_ref[...])


def _final_body(a0_ref, a1_ref, cnt_ref, t_ref, c_ref, i_ref, f_ref,
                wl_ref, wr_ref, b_ref, o_ref):
    s = a0_ref[...] + a1_ref[...]
    cnt = jnp.maximum(jnp.sum(cnt_ref[...], axis=0), 1.0)[:, None]
    mean = s / cnt
    t2 = (jnp.dot(mean, wl_ref[...], preferred_element_type=jnp.float32)
          + jnp.dot(t_ref[...], wr_ref[...],
                    preferred_element_type=jnp.float32)
          + b_ref[...])
    o_ref[...] = f_ref[...] * c_ref[...] + i_ref[...] * jnp.tanh(t2)


def kernel(x, edge_index, h, c, i, f, W_lin, b_node, Wl1, Wr1, b1, Wl2, Wr2, b2):
    n = x.shape[0]
    e = edge_index.shape[1]
    src = edge_index[0]
    dst = edge_index[1]
    grid = (-(-n // ROWB),)

    seg_sum1, npad = _make_seg_sum(n, e, with_cnt=True)
    seg_sum2, _ = _make_seg_sum(n, e, with_cnt=False)

    t0 = pl.pallas_call(
        _lin0_body,
        grid=grid,
        in_specs=[_row_spec(D), _full_spec((D, D))],
        out_specs=_row_spec(D),
        out_shape=jax.ShapeDtypeStruct((n, D), jnp.float32),
    )(x, W_lin.T)

    agg1, cnt = seg_sum1(t0, src, dst)

    cnt_spec = pl.BlockSpec((NW, ROWB), lambda i: (0, i))

    t1 = pl.pallas_call(
        _sage_body,
        grid=grid,
        in_specs=[_row_spec(D), _row_spec(D), cnt_spec, _row_spec(D),
                  _full_spec((D, D)), _full_spec((D, D)), _full_spec((1, D))],
        out_specs=_row_spec(D),
        out_shape=jax.ShapeDtypeStruct((n, D), jnp.float32),
    )(agg1[0], agg1[1], cnt, t0, Wl1.T, Wr1.T, b1.reshape(1, D))

    (agg2,) = seg_sum2(t1, src, dst)

    out = pl.pallas_call(
        _final_body,
        grid=grid,
        in_specs=[_row_spec(D), _row_spec(D), cnt_spec, _row_spec(D),
                  _row_spec(D), _row_spec(D), _row_spec(D),
                  _full_spec((D, D)), _full_spec((D, D)), _full_spec((1, D))],
        out_specs=_row_spec(D),
        out_shape=jax.ShapeDtypeStruct((n, D), jnp.float32),
    )(agg2[0], agg2[1], cnt, t1, c, i, f,
      Wl2.T, Wr2.T, (b2 + b_node).reshape(1, D))

    return out
